# baseline (speedup 1.0000x reference)
import jax
import jax.numpy as jnp
from jax import lax
from jax.experimental import pallas as pl
from jax.experimental.pallas import tpu as pltpu
from jax.experimental.pallas import tpu_sc as plsc

_V = 1000000
_B = 16384
_K = 20
_D = 16
_W = _D // 2
_L = 16
_NC = 2
_NS = 16
_NW = _NC * _NS
_BPW = _B // _NW
_CB = 128
_NCHUNK = _BPW // _CB
_NIW = 128
_NJ = _CB * _K // _NIW
_NROW = 1 + _K


def _sc_body(cw_hbm, pw_hbm, nw_hbm, in_hbm, out_hbm, sc_hbm,
             cidx_v, pidx_v, nidx_v, crow_v, prow_v, nrow_v, scr_v, sem):
  c = lax.axis_index("c")
  s = lax.axis_index("s")
  wid = s * _NC + c
  base = wid * _BPW

  @pl.loop(0, _NCHUNK)
  def _chunk(ci):
    b0 = base + ci * _CB
    pltpu.sync_copy(cw_hbm.at[pl.ds(b0, _CB)], cidx_v)
    pltpu.sync_copy(pw_hbm.at[pl.ds(b0, _CB)], pidx_v)
    pltpu.sync_copy(nw_hbm.at[pl.ds(b0 * _K, _CB * _K)], nidx_v)
    copies = [
        pltpu.async_copy(in_hbm.at[cidx_v], crow_v, sem),
        pltpu.async_copy(out_hbm.at[pidx_v], prow_v, sem),
    ]
    for j in range(_NJ):
      copies.append(pltpu.async_copy(
          out_hbm.at[nidx_v.at[pl.ds(j * _NIW, _NIW)]],
          nrow_v.at[pl.ds(j * _NIW, _NIW)], sem))
    for cp in copies:
      cp.wait()

    @pl.loop(0, _CB // _L)
    def _group(g):
      row0 = g * _L
      lane = lax.iota(jnp.int32, 16)
      rowi = row0 + lane
      rowk = rowi * _K
      cols = [jnp.full((16,), j, dtype=jnp.int32) for j in range(_W)]

      def unpack(ref, ridx, j):
        w = plsc.load_gather(ref, [ridx, cols[j]])
        return plsc.unpack(plsc.bitcast(w, jnp.bfloat16),
                           format=plsc.PackFormat.INTERLEAVED)

      cc = []
      for j in range(_W):
        a, b = unpack(crow_v, rowi, j)
        cc.extend((a, b))

      def dots(ref, ridx):
        acc = None
        for j in range(_W):
          a, b = unpack(ref, ridx, j)
          t = cc[2 * j] * a + cc[2 * j + 1] * b
          acc = t if acc is None else acc + t
        return acc

      scr_v[0, pl.ds(row0, _L)] = dots(prow_v, rowi)
      for k in range(_K):
        scr_v[1 + k, pl.ds(row0, _L)] = -dots(nrow_v, rowk + k)

    pltpu.sync_copy(scr_v, sc_hbm.at[:, pl.ds(b0, _CB)])


_sc_gather = pl.kernel(
    _sc_body,
    out_type=jax.ShapeDtypeStruct((_NROW, _B), jnp.float32),
    mesh=plsc.VectorSubcoreMesh(core_axis_name="c", subcore_axis_name="s"),
    compiler_params=pltpu.CompilerParams(
        needs_layout_passes=False, use_tc_tiling_on_sc=False),
    scratch_types=[
        pltpu.VMEM((_CB,), jnp.int32),
        pltpu.VMEM((_CB,), jnp.int32),
        pltpu.VMEM((_CB * _K,), jnp.int32),
        pltpu.VMEM((_CB, _W), jnp.float32),
        pltpu.VMEM((_CB, _W), jnp.float32),
        pltpu.VMEM((_CB * _K, _W), jnp.float32),
        pltpu.VMEM((_NROW, _CB), jnp.float32),
        pltpu.SemaphoreType.DMA,
    ],
)


def _loss_body(s_ref, o_ref):
  x = s_ref[...]
  o_ref[0, 0] = -jnp.sum(jax.nn.log_sigmoid(x)) / _B


_loss_call = pl.pallas_call(
    _loss_body,
    out_shape=jax.ShapeDtypeStruct((1, 1), jnp.float32),
    out_specs=pl.BlockSpec(memory_space=pltpu.SMEM),
)


def _pack(table):
  half = table.astype(jnp.bfloat16).reshape(_V, _W, 2)
  return lax.bitcast_convert_type(half, jnp.float32)


def kernel(center_words, pos_context_words, neg_context_words, in_embed, out_embed):
  cw = center_words.astype(jnp.int32)
  pw = pos_context_words.astype(jnp.int32)
  nw = neg_context_words.astype(jnp.int32).reshape(_B * _K)
  scores = _sc_gather(cw, pw, nw, _pack(in_embed), _pack(out_embed))
  return _loss_call(scores).reshape(())


# speedup vs baseline: 2.0315x; 2.0315x over previous
import jax
import jax.numpy as jnp
from jax import lax
from jax.experimental import pallas as pl
from jax.experimental.pallas import tpu as pltpu
from jax.experimental.pallas import tpu_sc as plsc

_V = 1000000
_B = 16384
_K = 20
_D = 16
_L = 16
_NC = 2
_NS = 16
_NW = _NC * _NS
_BPW = _B // _NW
_CB = 32
_NCHUNK = _BPW // _CB
_NIW = 128
_NJ = _CB * _K // _NIW
_NROW = 1 + _K
_SCHUNK = _NROW * _CB


def _sc_body(cw_hbm, pw_hbm, nw_hbm, in_hbm, out_hbm, sc_hbm,
             cidx_v, pidx_v, nidx_v, cg_v, pg_v, ng_v,
             crow_v, prow_v, nrow_v, scr_v, sem):
  c = lax.axis_index("c")
  s = lax.axis_index("s")
  wid = s * _NC + c
  base = wid * _BPW

  @pl.loop(0, _NCHUNK)
  def _chunk(ci):
    b0 = base + ci * _CB
    pltpu.sync_copy(cw_hbm.at[pl.ds(b0, _CB)], cidx_v)
    pltpu.sync_copy(pw_hbm.at[pl.ds(b0, _CB)], pidx_v)
    pltpu.sync_copy(nw_hbm.at[pl.ds(b0 * _K, _CB * _K)], nidx_v)
    for t in range(_CB // _L):
      cg_v[pl.ds(t * _L, _L)] = cidx_v[pl.ds(t * _L, _L)] >> 3
      pg_v[pl.ds(t * _L, _L)] = pidx_v[pl.ds(t * _L, _L)] >> 3
    for t in range(_CB * _K // _L):
      ng_v[pl.ds(t * _L, _L)] = nidx_v[pl.ds(t * _L, _L)] >> 3
    copies = [
        pltpu.async_copy(in_hbm.at[cg_v], crow_v, sem),
        pltpu.async_copy(out_hbm.at[pg_v], prow_v, sem),
    ]
    for j in range(_NJ):
      copies.append(pltpu.async_copy(
          out_hbm.at[ng_v.at[pl.ds(j * _NIW, _NIW)]],
          nrow_v.at[pl.ds(j * _NIW, _NIW)], sem))
    for cp in copies:
      cp.wait()

    @pl.loop(0, _CB // _L)
    def _group(g):
      row0 = g * _L
      lane = lax.iota(jnp.int32, 16)
      rowi = row0 + lane
      rowk = rowi * _K
      csub = (cidx_v[pl.ds(row0, _L)] & 7) * _D
      psub = (pidx_v[pl.ds(row0, _L)] & 7) * _D
      ccols = [plsc.load_gather(crow_v, [rowi, csub + d]) for d in range(_D)]

      acc = ccols[0] * plsc.load_gather(prow_v, [rowi, psub])
      for d in range(1, _D):
        acc = acc + ccols[d] * plsc.load_gather(prow_v, [rowi, psub + d])
      scr_v[pl.ds(row0, _L)] = acc

      for k in range(_K):
        ri = rowk + k
        nsub = plsc.load_gather(nidx_v, [ri])
        nsub = (nsub & 7) * _D
        acc = ccols[0] * plsc.load_gather(nrow_v, [ri, nsub])
        for d in range(1, _D):
          acc = acc + ccols[d] * plsc.load_gather(nrow_v, [ri, nsub + d])
        scr_v[pl.ds((1 + k) * _CB + row0, _L)] = -acc

    pltpu.sync_copy(scr_v, sc_hbm.at[pl.ds((wid * _NCHUNK + ci) * _SCHUNK,
                                           _SCHUNK)])


_sc_gather = pl.kernel(
    _sc_body,
    out_type=jax.ShapeDtypeStruct((_NROW * _B,), jnp.float32),
    mesh=plsc.VectorSubcoreMesh(core_axis_name="c", subcore_axis_name="s"),
    compiler_params=pltpu.CompilerParams(
        needs_layout_passes=False, use_tc_tiling_on_sc=True),
    scratch_types=[
        pltpu.VMEM((_CB,), jnp.int32),
        pltpu.VMEM((_CB,), jnp.int32),
        pltpu.VMEM((_CB * _K,), jnp.int32),
        pltpu.VMEM((_CB,), jnp.int32),
        pltpu.VMEM((_CB,), jnp.int32),
        pltpu.VMEM((_CB * _K,), jnp.int32),
        pltpu.VMEM((_CB, 128), jnp.float32),
        pltpu.VMEM((_CB, 128), jnp.float32),
        pltpu.VMEM((_CB * _K, 128), jnp.float32),
        pltpu.VMEM((_SCHUNK,), jnp.float32),
        pltpu.SemaphoreType.DMA,
    ],
)


def _loss_body(s_ref, o_ref):
  x = s_ref[...]
  o_ref[0, 0] = -jnp.sum(jax.nn.log_sigmoid(x)) / _B


_loss_call = pl.pallas_call(
    _loss_body,
    out_shape=jax.ShapeDtypeStruct((1, 1), jnp.float32),
    out_specs=pl.BlockSpec(memory_space=pltpu.SMEM),
)


def kernel(center_words, pos_context_words, neg_context_words, in_embed, out_embed):
  cw = center_words.astype(jnp.int32)
  pw = pos_context_words.astype(jnp.int32)
  nw = neg_context_words.astype(jnp.int32).reshape(_B * _K)
  scores = _sc_gather(cw, pw, nw,
                      in_embed.reshape(_V // 8, 128),
                      out_embed.reshape(_V // 8, 128))
  return _loss_call(scores.reshape(_NROW * _B // 128, 128)).reshape(())
